# baseline (speedup 1.0000x reference)
import functools

import jax
import jax.numpy as jnp
from jax import lax
from jax.experimental import pallas as pl
from jax.experimental.pallas import tpu as pltpu
from jax.experimental.pallas import tpu_sc as plsc

BATCH = 4096
CTX = 50
EMBED_DIM = 64
OUTPUT_DIM = 1000

NUM_CORES = 2
NUM_SUBCORES = 16
NUM_WORKERS = NUM_CORES * NUM_SUBCORES
ROWS_PER_WORKER = BATCH // NUM_WORKERS
NLANE = 16
DREG = EMBED_DIM // NLANE


def _sc_pool(idx_hbm, table_hbm, out_hbm, idx_v, rows_v, acc_v, sem):
    wid = lax.axis_index("s") * NUM_CORES + lax.axis_index("c")
    base = wid * ROWS_PER_WORKER

    pltpu.sync_copy(idx_hbm.at[pl.ds(base, ROWS_PER_WORKER)], idx_v)

    def per_row(r, carry):
        pltpu.async_copy(table_hbm.at[idx_v.at[r]], rows_v, sem).wait()

        def body(j, accs):
            return tuple(
                accs[d] + rows_v[j, pl.ds(d * NLANE, NLANE)]
                for d in range(DREG)
            )

        zeros = tuple(jnp.zeros((NLANE,), jnp.float32) for _ in range(DREG))
        accs = lax.fori_loop(0, CTX, body, zeros)
        for d in range(DREG):
            acc_v[r, pl.ds(d * NLANE, NLANE)] = accs[d]
        return carry

    lax.fori_loop(0, ROWS_PER_WORKER, per_row, 0)

    pltpu.sync_copy(acc_v, out_hbm.at[pl.ds(base, ROWS_PER_WORKER)])


def _pool_embeddings(idx, table):
    mesh = plsc.VectorSubcoreMesh(core_axis_name="c", subcore_axis_name="s")
    kern = functools.partial(
        pl.kernel,
        mesh=mesh,
        out_type=jax.ShapeDtypeStruct((BATCH, EMBED_DIM), jnp.float32),
        scratch_types=[
            pltpu.VMEM((ROWS_PER_WORKER, CTX), jnp.int32),
            pltpu.VMEM((CTX, EMBED_DIM), jnp.float32),
            pltpu.VMEM((ROWS_PER_WORKER, EMBED_DIM), jnp.float32),
            pltpu.SemaphoreType.DMA,
        ],
        compiler_params=pltpu.CompilerParams(use_tc_tiling_on_sc=False),
    )(_sc_pool)
    return kern(idx, table)


def _matmul_body(x_ref, w_ref, bvec_ref, o_ref):
    x = x_ref[...]
    w = w_ref[...]
    acc = lax.dot_general(
        x, w,
        dimension_numbers=(((1,), (1,)), ((), ())),
        preferred_element_type=jnp.float32,
        precision=lax.Precision.HIGHEST,
    )
    o_ref[...] = acc + bvec_ref[...]


def _linear(x, W, bvec):
    blk = 512
    grid = (BATCH // blk,)
    return pl.pallas_call(
        _matmul_body,
        grid=grid,
        in_specs=[
            pl.BlockSpec((blk, EMBED_DIM), lambda i: (i, 0)),
            pl.BlockSpec((OUTPUT_DIM, EMBED_DIM), lambda i: (0, 0)),
            pl.BlockSpec((1, OUTPUT_DIM), lambda i: (0, 0)),
        ],
        out_specs=pl.BlockSpec((blk, OUTPUT_DIM), lambda i: (i, 0)),
        out_shape=jax.ShapeDtypeStruct((BATCH, OUTPUT_DIM), jnp.float32),
    )(x, W, bvec)


def kernel(inputs, embed_table, W, b, bias):
    idx = inputs.astype(jnp.int32)
    pooled = _pool_embeddings(idx, embed_table)
    bvec = (b + bias).astype(jnp.float32).reshape(1, OUTPUT_DIM)
    return _linear(pooled, W, bvec)


# speedup vs baseline: 1.1389x; 1.1389x over previous
import functools

import jax
import jax.numpy as jnp
from jax import lax
from jax.experimental import pallas as pl
from jax.experimental.pallas import tpu as pltpu
from jax.experimental.pallas import tpu_sc as plsc

BATCH = 4096
CTX = 50
EMBED_DIM = 64
OUTPUT_DIM = 1000

NUM_CORES = 2
NUM_SUBCORES = 16
NUM_WORKERS = NUM_CORES * NUM_SUBCORES
ROWS_PER_WORKER = BATCH // NUM_WORKERS
NLANE = 16
DREG = EMBED_DIM // NLANE


NBUF = 8


def _sc_pool(idx_hbm, table_hbm, out_hbm, idx_v, acc_v, *bufs_and_sems):
    bufs = bufs_and_sems[:NBUF]
    sems = bufs_and_sems[NBUF:]
    wid = lax.axis_index("s") * NUM_CORES + lax.axis_index("c")
    base = wid * ROWS_PER_WORKER

    pltpu.sync_copy(idx_hbm.at[pl.ds(base, ROWS_PER_WORKER)], idx_v)

    for b in range(NBUF):
        pltpu.async_copy(table_hbm.at[idx_v.at[b]], bufs[b], sems[b])

    def accumulate(buf, r):
        def body(j, accs):
            return tuple(
                accs[d]
                + buf[2 * j, pl.ds(d * NLANE, NLANE)]
                + buf[2 * j + 1, pl.ds(d * NLANE, NLANE)]
                for d in range(DREG)
            )

        zeros = tuple(jnp.zeros((NLANE,), jnp.float32) for _ in range(DREG))
        accs = lax.fori_loop(0, CTX // 2, body, zeros)
        for d in range(DREG):
            acc_v[r, pl.ds(d * NLANE, NLANE)] = accs[d]

    def group(go, carry):
        for b in range(NBUF):
            r = go * NBUF + b
            pltpu.make_async_copy(
                table_hbm.at[idx_v.at[r]], bufs[b], sems[b]
            ).wait()
            accumulate(bufs[b], r)

            @pl.when(r + NBUF < ROWS_PER_WORKER)
            def _():
                pltpu.async_copy(
                    table_hbm.at[idx_v.at[r + NBUF]], bufs[b], sems[b]
                )

        return carry

    lax.fori_loop(0, ROWS_PER_WORKER // NBUF, group, 0)

    pltpu.sync_copy(acc_v, out_hbm.at[pl.ds(base, ROWS_PER_WORKER)])


def _pool_embeddings(idx, table):
    mesh = plsc.VectorSubcoreMesh(core_axis_name="c", subcore_axis_name="s")
    kern = functools.partial(
        pl.kernel,
        mesh=mesh,
        out_type=jax.ShapeDtypeStruct((BATCH, EMBED_DIM), jnp.float32),
        scratch_types=(
            [
                pltpu.VMEM((ROWS_PER_WORKER, CTX), jnp.int32),
                pltpu.VMEM((ROWS_PER_WORKER, EMBED_DIM), jnp.float32),
            ]
            + [pltpu.VMEM((CTX, EMBED_DIM), jnp.float32)] * NBUF
            + [pltpu.SemaphoreType.DMA] * NBUF
        ),
        compiler_params=pltpu.CompilerParams(use_tc_tiling_on_sc=False),
    )(_sc_pool)
    return kern(idx, table)


def _matmul_body(x_ref, w_ref, bvec_ref, o_ref):
    x = x_ref[...]
    w = w_ref[...]
    acc = lax.dot_general(
        x, w,
        dimension_numbers=(((1,), (1,)), ((), ())),
        preferred_element_type=jnp.float32,
        precision=lax.Precision.HIGHEST,
    )
    o_ref[...] = acc + bvec_ref[...]


def _linear(x, W, bvec):
    blk = 512
    grid = (BATCH // blk,)
    return pl.pallas_call(
        _matmul_body,
        grid=grid,
        in_specs=[
            pl.BlockSpec((blk, EMBED_DIM), lambda i: (i, 0)),
            pl.BlockSpec((OUTPUT_DIM, EMBED_DIM), lambda i: (0, 0)),
            pl.BlockSpec((1, OUTPUT_DIM), lambda i: (0, 0)),
        ],
        out_specs=pl.BlockSpec((blk, OUTPUT_DIM), lambda i: (i, 0)),
        out_shape=jax.ShapeDtypeStruct((BATCH, OUTPUT_DIM), jnp.float32),
    )(x, W, bvec)


def kernel(inputs, embed_table, W, b, bias):
    idx = inputs.astype(jnp.int32)
    pooled = _pool_embeddings(idx, embed_table)
    bvec = (b + bias).astype(jnp.float32).reshape(1, OUTPUT_DIM)
    return _linear(pooled, W, bvec)
